# speedup vs baseline: 3.6275x; 3.6275x over previous
import jax
import jax.numpy as jnp
from jax import lax
from jax.experimental import pallas as pl
from jax.experimental.pallas import tpu as pltpu
from jax.experimental.pallas import tpu_sc as plsc

_N = 131072
_ZROWS = 86
_D = 256

_NC = 2
_NS = 16
_NW = _NC * _NS
_BPW = _N // _NW
_C = 128
_NCHUNK = _BPW // _C


def _table_body(ec_ref, w_ref, zt_ref, out_ref):
    out_ref[...] = (
        jnp.dot(ec_ref[...], w_ref[...], preferred_element_type=jnp.float32)
        + zt_ref[...]
    )


def _fused_table(ec86, w, zt):
    return pl.pallas_call(
        _table_body,
        out_shape=jax.ShapeDtypeStruct((_ZROWS, _D), jnp.float32),
    )(ec86, w, zt)


def _gather_body(table_hbm, idx_hbm, out_hbm,
                 idx_v, buf0, buf1, gsem0, gsem1, osem0, osem1):
    wid = lax.axis_index("s") * _NC + lax.axis_index("c")
    base = wid * _BPW
    pltpu.sync_copy(idx_hbm.at[pl.ds(base, _BPW)], idx_v)

    bufs = (buf0, buf1)
    gsems = (gsem0, gsem1)
    osems = (osem0, osem1)

    for b in range(2):
        pltpu.async_copy(table_hbm.at[idx_v.at[pl.ds(b * _C, _C)]],
                         bufs[b], gsems[b])

    @pl.loop(0, _NCHUNK, step=2)
    def _chunks(g):
        for b in range(2):
            gi = g + b
            row0 = base + gi * _C
            pltpu.make_async_copy(out_hbm.at[pl.ds(row0, _C)],
                                  bufs[b], gsems[b]).wait()
            pltpu.async_copy(bufs[b], out_hbm.at[pl.ds(row0, _C)], osems[b])
            pltpu.make_async_copy(bufs[b], out_hbm.at[pl.ds(row0, _C)],
                                  osems[b]).wait()

            @pl.when(gi + 2 < _NCHUNK)
            def _():
                pltpu.async_copy(
                    table_hbm.at[idx_v.at[pl.ds((gi + 2) * _C, _C)]],
                    bufs[b], gsems[b])


def kernel(z, elec_config, m_weight, z_table):
    zi = z.astype(jnp.int32)
    table = _fused_table(elec_config[:_ZROWS], m_weight, z_table)
    mesh = plsc.VectorSubcoreMesh(core_axis_name="c", subcore_axis_name="s",
                                  num_cores=_NC, num_subcores=_NS)
    gather = pl.kernel(
        _gather_body,
        out_type=jax.ShapeDtypeStruct((_N, _D), jnp.float32),
        mesh=mesh,
        scratch_types=[
            pltpu.VMEM((_BPW,), jnp.int32),
            pltpu.VMEM((_C, _D), jnp.float32),
            pltpu.VMEM((_C, _D), jnp.float32),
            pltpu.SemaphoreType.DMA,
            pltpu.SemaphoreType.DMA,
            pltpu.SemaphoreType.DMA,
            pltpu.SemaphoreType.DMA,
        ],
    )
    return gather(table, zi)
